# baseline (speedup 1.0000x reference)
import dataclasses

import jax
import jax.numpy as jnp
from jax import lax
from jax.experimental import pallas as pl
from jax.experimental.pallas import tpu as pltpu
from jax.experimental.pallas import tpu_sc as plsc

N = 10000
E = 160000
K = 30

NC = 2
NS = 16

EPW1 = E // (NC * NS)
EPW1_PAD = 5008
N4 = 40960
NPASS = 20

CH = 128
NCH = 80
EPS2 = CH * NCH
E_PAD = EPS2 * NS

RPS = 624
WB_BLOCKS = (128, 128, 128, 128, 112)

_mesh = plsc.VectorSubcoreMesh(core_axis_name="c", subcore_axis_name="s")

_sc_params = pltpu.CompilerParams()
if "needs_layout_passes" in pltpu.CompilerParams.__dataclass_fields__:
    _sc_params = dataclasses.replace(_sc_params, needs_layout_passes=False)


def _sc1_body(colors_hbm, probas_hbm, src_hbm, dst_hbm, w0_hbm,
              w_out, agg0_out,
              colors_v, probas_v, src_v, dst_v, w_v, w0_v, acc_v,
              red_a, red_in, stage):
    c = lax.axis_index("c")
    s = lax.axis_index("s")
    wid = c * NS + s

    pltpu.sync_copy(colors_hbm, colors_v)
    pltpu.sync_copy(probas_hbm, probas_v)
    pltpu.sync_copy(w0_hbm, w0_v)
    base_e = wid * EPW1
    pltpu.sync_copy(src_hbm.at[pl.ds(base_e, EPW1)], src_v.at[pl.ds(0, EPW1)])
    pltpu.sync_copy(dst_hbm.at[pl.ds(base_e, EPW1)], dst_v.at[pl.ds(0, EPW1)])

    zero16 = jnp.zeros((16,), jnp.float32)

    @pl.loop(0, N4, step=16)
    def _zero(i):
        acc_v[pl.ds(i, 16)] = zero16

    nmax = jnp.full((16,), N - 1, jnp.int32)
    izero = jnp.zeros((16,), jnp.int32)
    lane = lax.iota(jnp.int32, 16)
    half = jnp.full((16,), 0.5, jnp.float32)

    w0vec = w0_v[pl.ds(0, 16)]
    w00 = w0vec[0]
    w01 = w0vec[1]
    w02 = w0vec[2]
    w10 = w0vec[3]
    w11 = w0vec[4]
    w12 = w0vec[5]
    w20 = w0vec[6]
    w21 = w0vec[7]
    w22 = w0vec[8]

    @pl.loop(0, EPW1_PAD, step=16)
    def _edges(b):
        valid = (lane + b) < EPW1
        src16 = jnp.minimum(jnp.maximum(src_v[pl.ds(b, 16)], izero), nmax)
        dst16 = jnp.minimum(jnp.maximum(dst_v[pl.ds(b, 16)], izero), nmax)
        ps = plsc.load_gather(probas_v, [src16])
        pd = plsc.load_gather(probas_v, [dst16])
        m = ((ps >= half) & (pd >= half)) | ((ps < half) & (pd < half))
        cs0 = plsc.load_gather(colors_v, [src16])
        cd0 = plsc.load_gather(colors_v, [dst16])
        cs1 = plsc.load_gather(colors_v, [src16 + N])
        cd1 = plsc.load_gather(colors_v, [dst16 + N])
        cs2 = plsc.load_gather(colors_v, [src16 + 2 * N])
        cd2 = plsc.load_gather(colors_v, [dst16 + 2 * N])
        d0 = cs0 - cd0
        d1 = cs1 - cd1
        d2c = cs2 - cd2
        dist = d0 * d0 + d1 * d1 + d2c * d2c
        s0 = jnp.exp(dist * jnp.float32(-1.0 / 255.0))
        wv = jnp.where(m, s0, jnp.float32(0.0))
        w_v[pl.ds(b, 16)] = wv
        h0a = cs0 * w00 + cs1 * w10 + cs2 * w20
        h0b = cs0 * w01 + cs1 * w11 + cs2 * w21
        h0c = cs0 * w02 + cs1 * w12 + cs2 * w22
        di = dst16 * 4
        plsc.addupdate_scatter(acc_v, [di], wv * h0a, mask=valid)
        plsc.addupdate_scatter(acc_v, [di + 1], wv * h0b, mask=valid)
        plsc.addupdate_scatter(acc_v, [di + 2], wv * h0c, mask=valid)

    pltpu.sync_copy(w_v.at[pl.ds(0, EPW1)], w_out.at[pl.ds(base_e, EPW1)])

    part_sz = N4 // NPASS
    seg = part_sz // NS
    off = s * seg
    for part in range(NPASS):
        pltpu.sync_copy(acc_v.at[pl.ds(part * part_sz, part_sz)],
                        stage.at[s])
        plsc.subcore_barrier()
        pltpu.sync_copy(stage.at[:, pl.ds(off, seg)], red_in)

        @pl.loop(0, seg, step=16)
        def _add(i):
            tot = red_in[0, pl.ds(i, 16)]
            for p in range(1, NS):
                tot = tot + red_in[p, pl.ds(i, 16)]
            red_a[pl.ds(i, 16)] = tot

        pltpu.sync_copy(red_a,
                        agg0_out.at[c, pl.ds(part * part_sz + off, seg)])
        plsc.subcore_barrier()


def _sc1(colors_flat, probas, src, dst, w0_pad):
    kern = pl.kernel(
        _sc1_body,
        out_type=[
            jax.ShapeDtypeStruct((E,), jnp.float32),
            jax.ShapeDtypeStruct((NC, N4), jnp.float32),
        ],
        mesh=_mesh,
        scratch_types=[
            pltpu.VMEM((3 * N,), jnp.float32),
            pltpu.VMEM((N,), jnp.float32),
            pltpu.VMEM((EPW1_PAD,), jnp.int32),
            pltpu.VMEM((EPW1_PAD,), jnp.int32),
            pltpu.VMEM((EPW1_PAD,), jnp.float32),
            pltpu.VMEM((16,), jnp.float32),
            pltpu.VMEM((N4,), jnp.float32),
            pltpu.VMEM((N4 // NPASS // NS,), jnp.float32),
            pltpu.VMEM((NS, N4 // NPASS // NS), jnp.float32),
            pltpu.VMEM_SHARED((NS, N4 // NPASS), jnp.float32),
        ],
        compiler_params=_sc_params,
    )
    return kern(colors_flat, probas, src, dst, w0_pad)


def _sc2_half(h1x_hbm, outx_hbm, w_hbm, dst_hbm, src_v, wbuf, dbuf,
              rows, rows2, acc, gsem, msem, s):
    zero16 = jnp.zeros((16,), jnp.float32)

    @pl.loop(0, CH)
    def _zz(i):
        for t in range(8):
            rows[i, pl.ds(t * 16, 16)] = zero16

    row0 = s * RPS
    off = 0
    for blk in WB_BLOCKS:
        pltpu.sync_copy(rows.at[pl.ds(0, blk)],
                        acc.at[pl.ds(row0 + off, blk)])
        off += blk

    @pl.when(s == NS - 1)
    def _ztail():
        pltpu.sync_copy(rows.at[pl.ds(0, 16)],
                        acc.at[pl.ds(NS * RPS, 16)])

    plsc.subcore_barrier()

    base_e = s * EPS2

    def _fetch_start(k, b, buf):
        pltpu.make_async_copy(
            h1x_hbm.at[src_v.at[pl.ds(k * CH, CH)]], buf, gsem).start()
        pltpu.make_async_copy(
            w_hbm.at[pl.ds(base_e + k * CH, CH)], wbuf.at[b], msem).start()
        pltpu.make_async_copy(dst_hbm.at[s, k], dbuf.at[b], msem).start()

    def _fetch_wait(b, buf):
        pltpu.make_async_copy(
            h1x_hbm.at[src_v.at[pl.ds(0, CH)]], buf, gsem).wait()
        pltpu.make_async_copy(
            w_hbm.at[pl.ds(0, CH)], wbuf.at[b], msem).wait()
        pltpu.make_async_copy(dst_hbm.at[0, 0], dbuf.at[b], msem).wait()

    def _scale_scatter(b, buf):
        @pl.loop(0, CH, step=16)
        def _scale(g):
            wgrp = wbuf[b, pl.ds(g, 16)]
            for j in range(16):
                wj = wgrp[j]
                for t in range(8):
                    buf[g + j, pl.ds(t * 16, 16)] = (
                        buf[g + j, pl.ds(t * 16, 16)] * wj)

        pltpu.sync_copy(buf, acc.at[dbuf.at[b]], add=True)

    _fetch_start(0, 0, rows)

    @pl.loop(0, NCH, step=2)
    def _chunk(k):
        _fetch_start(k + 1, 1, rows2)
        _fetch_wait(0, rows)
        _scale_scatter(0, rows)

        @pl.when(k + 2 < NCH)
        def _():
            _fetch_start(k + 2, 0, rows)

        _fetch_wait(1, rows2)
        _scale_scatter(1, rows2)

    plsc.subcore_barrier()
    off = 0
    for blk in WB_BLOCKS:
        r = row0 + off
        pltpu.sync_copy(acc.at[pl.ds(r, blk)], outx_hbm.at[pl.ds(r, blk)])
        off += blk

    @pl.when(s == NS - 1)
    def _wtail():
        pltpu.sync_copy(acc.at[pl.ds(NS * RPS, 16)],
                        outx_hbm.at[pl.ds(NS * RPS, 16)])


def _sc2_body(h1a_hbm, h1b_hbm, src_hbm, dst_hbm, w_hbm,
              z2a_hbm, z2b_hbm,
              src_v, wbuf, dbuf, rows, rows2, acc, gsem, msem):
    c = lax.axis_index("c")
    s = lax.axis_index("s")
    base_e = s * EPS2
    pltpu.sync_copy(src_hbm.at[pl.ds(base_e, EPS2)], src_v)

    @pl.when(c == 0)
    def _():
        _sc2_half(h1a_hbm, z2a_hbm, w_hbm, dst_hbm, src_v, wbuf, dbuf,
                  rows, rows2, acc, gsem, msem, s)

    @pl.when(c == 1)
    def _():
        _sc2_half(h1b_hbm, z2b_hbm, w_hbm, dst_hbm, src_v, wbuf, dbuf,
                  rows, rows2, acc, gsem, msem, s)


def _sc2(h1a, h1b, src_pad, dst_resh, w_pad):
    kern = pl.kernel(
        _sc2_body,
        out_type=[
            jax.ShapeDtypeStruct((N, 128), jnp.float32),
            jax.ShapeDtypeStruct((N, 128), jnp.float32),
        ],
        mesh=_mesh,
        scratch_types=[
            pltpu.VMEM((EPS2,), jnp.int32),
            pltpu.VMEM((2, CH), jnp.float32),
            pltpu.VMEM((2, CH), jnp.int32),
            pltpu.VMEM((CH, 128), jnp.float32),
            pltpu.VMEM((CH, 128), jnp.float32),
            pltpu.VMEM_SHARED((N, 128), jnp.float32),
            pltpu.SemaphoreType.DMA,
            pltpu.SemaphoreType.DMA,
        ],
        compiler_params=_sc_params,
    )
    return kern(h1a, h1b, src_pad, dst_resh, w_pad)


def _g_body(feats_ref, w1b_ref, out_ref):
    out_ref[...] = jax.lax.dot_general(
        feats_ref[...], w1b_ref[...], (((1,), (0,)), ((), ())),
        preferred_element_type=jnp.float32,
        precision=jax.lax.Precision.HIGHEST)


def _tc_g(feats, w1b):
    bn = 1000
    return pl.pallas_call(
        _g_body,
        grid=(N // bn,),
        in_specs=[
            pl.BlockSpec((bn, 512), lambda i: (i, 0)),
            pl.BlockSpec((512, 256), lambda i: (0, 0)),
        ],
        out_specs=pl.BlockSpec((bn, 256), lambda i: (i, 0)),
        out_shape=jax.ShapeDtypeStruct((N, 256), jnp.float32),
    )(feats, w1b)


def _b_body(agg_ref, colors_ref, g_ref, w0_ref, b0_ref, w1a_ref,
            h1a_ref, h1b_ref):
    agg = agg_ref[0, :, 0:3] + agg_ref[1, :, 0:3]
    h0 = jax.lax.dot_general(
        colors_ref[...], w0_ref[...], (((1,), (0,)), ((), ())),
        preferred_element_type=jnp.float32,
        precision=jax.lax.Precision.HIGHEST)
    r = jnp.maximum(agg + h0 + b0_ref[...], 0.0)
    h1 = jax.lax.dot_general(
        r, w1a_ref[...], (((1,), (0,)), ((), ())),
        preferred_element_type=jnp.float32,
        precision=jax.lax.Precision.HIGHEST) + g_ref[...]
    h1a_ref[...] = h1[:, :128]
    h1b_ref[...] = h1[:, 128:]


def _tc_b(agg0, colors, g, w0, b0_row, w1a):
    bn = 1000
    agg0_r = agg0.reshape(NC, N4 // 4, 4)
    return pl.pallas_call(
        _b_body,
        grid=(N // bn,),
        in_specs=[
            pl.BlockSpec((NC, bn, 4), lambda i: (0, i, 0)),
            pl.BlockSpec((bn, 3), lambda i: (i, 0)),
            pl.BlockSpec((bn, 256), lambda i: (i, 0)),
            pl.BlockSpec((3, 3), lambda i: (0, 0)),
            pl.BlockSpec((1, 3), lambda i: (0, 0)),
            pl.BlockSpec((3, 256), lambda i: (0, 0)),
        ],
        out_specs=[
            pl.BlockSpec((bn, 128), lambda i: (i, 0)),
            pl.BlockSpec((bn, 128), lambda i: (i, 0)),
        ],
        out_shape=[
            jax.ShapeDtypeStruct((N, 128), jnp.float32),
            jax.ShapeDtypeStruct((N, 128), jnp.float32),
        ],
    )(agg0_r, colors, g, w0, b0_row, w1a)


def _c_body(z2a_ref, z2b_ref, h1a_ref, h1b_ref, aspp_ref, b1_ref, mu_ref,
            out_ref):
    agg = jnp.concatenate([z2a_ref[...], z2b_ref[...]], axis=1)
    h1 = jnp.concatenate([h1a_ref[...], h1b_ref[...]], axis=1)
    z2 = 0.5 * (agg + h1 + b1_ref[...] + aspp_ref[...])
    mu = mu_ref[...]
    dot = jax.lax.dot_general(
        z2, mu, (((1,), (1,)), ((), ())),
        preferred_element_type=jnp.float32,
        precision=jax.lax.Precision.HIGHEST)
    z2sq = jnp.sum(z2 * z2, axis=1, keepdims=True)
    musq = jnp.sum(mu * mu, axis=1)[None, :]
    d2 = jnp.maximum(z2sq + musq - 2.0 * dot, 0.0)
    f2 = 1.0 / (1.0 + d2)
    fmax = jnp.max(f2, axis=1, keepdims=True)
    ef = jnp.exp(f2 - fmax)
    out_ref[...] = ef / jnp.sum(ef, axis=1, keepdims=True)


def _tc_c(z2a, z2b, h1a, h1b, aspp, b1_row, mu):
    bn = 1000
    return pl.pallas_call(
        _c_body,
        grid=(N // bn,),
        in_specs=[
            pl.BlockSpec((bn, 128), lambda i: (i, 0)),
            pl.BlockSpec((bn, 128), lambda i: (i, 0)),
            pl.BlockSpec((bn, 128), lambda i: (i, 0)),
            pl.BlockSpec((bn, 128), lambda i: (i, 0)),
            pl.BlockSpec((bn, 256), lambda i: (i, 0)),
            pl.BlockSpec((1, 256), lambda i: (0, 0)),
            pl.BlockSpec((K, 256), lambda i: (0, 0)),
        ],
        out_specs=pl.BlockSpec((bn, K), lambda i: (i, 0)),
        out_shape=jax.ShapeDtypeStruct((N, K), jnp.float32),
    )(z2a, z2b, h1a, h1b, aspp, b1_row, mu)


def kernel(nodes_color, probas, feats_pooled, pooled_aspp_feats, edges_nn,
           W0, b0, W1, b1, mu):
    src = edges_nn[:, 0]
    dst = edges_nn[:, 1]
    colors_flat = nodes_color.T.reshape(-1)
    w0_pad = jnp.pad(W0.reshape(-1), (0, 7))

    w_e, agg0 = _sc1(colors_flat, probas, src, dst, w0_pad)
    g = _tc_g(feats_pooled, W1[3:])
    h1a, h1b = _tc_b(agg0, nodes_color, g, W0, b0.reshape(1, 3), W1[:3])

    src_pad = jnp.pad(src, (0, E_PAD - E))
    dst_pad = jnp.pad(dst, (0, E_PAD - E)).reshape(NS, NCH, CH)
    w_pad = jnp.pad(w_e, (0, E_PAD - E))
    z2a, z2b = _sc2(h1a, h1b, src_pad, dst_pad, w_pad)

    return _tc_c(z2a, z2b, h1a, h1b, pooled_aspp_feats,
                 b1.reshape(1, 256), mu)


# speedup vs baseline: 1.3182x; 1.3182x over previous
import dataclasses

import jax
import jax.numpy as jnp
from jax import lax
from jax.experimental import pallas as pl
from jax.experimental.pallas import tpu as pltpu
from jax.experimental.pallas import tpu_sc as plsc

N = 10000
E = 160000
K = 30

NC = 2
NS = 16
NW = NC * NS

EPW1 = E // NW
EPW1_PAD = 5008
REG = 5120
N4 = 40960
NPASS = 10

CH = 128

RPS = 624
WB_BLOCKS = (128, 128, 128, 128, 112)

_mesh = plsc.VectorSubcoreMesh(core_axis_name="c", subcore_axis_name="s")

_sc_params = pltpu.CompilerParams()
if "needs_layout_passes" in pltpu.CompilerParams.__dataclass_fields__:
    _sc_params = dataclasses.replace(_sc_params, needs_layout_passes=False)


def _sc1_body(colors_hbm, probas_hbm, src_hbm, dst_hbm, w0_hbm,
              srcc_out, dstc_out, wc_out, counts_out, agg0_out,
              colors_v, probas_v, src_v, dst_v, w0_v, acc_v,
              cs_v, cd_v, cw_v, cnt_v, red_a, red_in, stage):
    c = lax.axis_index("c")
    s = lax.axis_index("s")
    wid = c * NS + s

    pltpu.sync_copy(colors_hbm, colors_v)
    pltpu.sync_copy(probas_hbm, probas_v)
    pltpu.sync_copy(w0_hbm, w0_v)
    base_e = wid * EPW1
    pltpu.sync_copy(src_hbm.at[pl.ds(base_e, EPW1)], src_v.at[pl.ds(0, EPW1)])
    pltpu.sync_copy(dst_hbm.at[pl.ds(base_e, EPW1)], dst_v.at[pl.ds(0, EPW1)])

    zero16 = jnp.zeros((16,), jnp.float32)
    izero16 = jnp.zeros((16,), jnp.int32)

    @pl.loop(0, N4, step=16)
    def _zero(i):
        acc_v[pl.ds(i, 16)] = zero16

    @pl.loop(0, REG, step=16)
    def _zeroc(i):
        cs_v[pl.ds(i, 16)] = izero16
        cd_v[pl.ds(i, 16)] = izero16
        cw_v[pl.ds(i, 16)] = zero16

    nmax = jnp.full((16,), N - 1, jnp.int32)
    lane = lax.iota(jnp.int32, 16)
    half = jnp.full((16,), 0.5, jnp.float32)

    w0vec = w0_v[pl.ds(0, 16)]
    w00 = w0vec[0]
    w01 = w0vec[1]
    w02 = w0vec[2]
    w10 = w0vec[3]
    w11 = w0vec[4]
    w12 = w0vec[5]
    w20 = w0vec[6]
    w21 = w0vec[7]
    w22 = w0vec[8]

    def _edge_body(i, ptr):
        b = i * 16
        valid = (lane + b) < EPW1
        src16 = jnp.minimum(jnp.maximum(src_v[pl.ds(b, 16)], izero16), nmax)
        dst16 = jnp.minimum(jnp.maximum(dst_v[pl.ds(b, 16)], izero16), nmax)
        ps = plsc.load_gather(probas_v, [src16])
        pd = plsc.load_gather(probas_v, [dst16])
        m = ((ps >= half) & (pd >= half)) | ((ps < half) & (pd < half))
        cs0 = plsc.load_gather(colors_v, [src16])
        cd0 = plsc.load_gather(colors_v, [dst16])
        cs1 = plsc.load_gather(colors_v, [src16 + N])
        cd1 = plsc.load_gather(colors_v, [dst16 + N])
        cs2 = plsc.load_gather(colors_v, [src16 + 2 * N])
        cd2 = plsc.load_gather(colors_v, [dst16 + 2 * N])
        d0 = cs0 - cd0
        d1 = cs1 - cd1
        d2c = cs2 - cd2
        dist = d0 * d0 + d1 * d1 + d2c * d2c
        s0 = jnp.exp(dist * jnp.float32(-1.0 / 255.0))
        wv = jnp.where(m, s0, jnp.float32(0.0))
        h0a = cs0 * w00 + cs1 * w10 + cs2 * w20
        h0b = cs0 * w01 + cs1 * w11 + cs2 * w21
        h0c = cs0 * w02 + cs1 * w12 + cs2 * w22
        di = dst16 * 4
        plsc.addupdate_scatter(acc_v, [di], wv * h0a, mask=valid)
        plsc.addupdate_scatter(acc_v, [di + 1], wv * h0b, mask=valid)
        plsc.addupdate_scatter(acc_v, [di + 2], wv * h0c, mask=valid)
        keep = m & valid
        plsc.store_compressed(cs_v.at[pl.ds(ptr, 16)], src16, mask=keep)
        plsc.store_compressed(cd_v.at[pl.ds(ptr, 16)], dst16, mask=keep)
        plsc.store_compressed(cw_v.at[pl.ds(ptr, 16)], wv, mask=keep)
        return ptr + plsc.all_reduce_population_count(keep)[0]

    nkept = lax.fori_loop(0, EPW1_PAD // 16, _edge_body, jnp.int32(0))

    pltpu.sync_copy(cs_v, srcc_out.at[wid])
    pltpu.sync_copy(cd_v, dstc_out.at[wid])
    pltpu.sync_copy(cw_v, wc_out.at[wid])
    cnt_v[pl.ds(0, 16)] = jnp.where(lane == 0, nkept, 0)
    pltpu.sync_copy(cnt_v, counts_out.at[wid])

    part_sz = N4 // NPASS
    seg = part_sz // NS
    off = s * seg
    for part in range(NPASS):
        pltpu.sync_copy(acc_v.at[pl.ds(part * part_sz, part_sz)],
                        stage.at[s])
        plsc.subcore_barrier()
        pltpu.sync_copy(stage.at[:, pl.ds(off, seg)], red_in)

        @pl.loop(0, seg, step=16)
        def _add(i):
            tot = red_in[0, pl.ds(i, 16)]
            for p in range(1, NS):
                tot = tot + red_in[p, pl.ds(i, 16)]
            red_a[pl.ds(i, 16)] = tot

        pltpu.sync_copy(red_a,
                        agg0_out.at[c, pl.ds(part * part_sz + off, seg)])
        plsc.subcore_barrier()


def _sc1(colors_flat, probas, src, dst, w0_pad):
    kern = pl.kernel(
        _sc1_body,
        out_type=[
            jax.ShapeDtypeStruct((NW, REG), jnp.int32),
            jax.ShapeDtypeStruct((NW, REG), jnp.int32),
            jax.ShapeDtypeStruct((NW, REG), jnp.float32),
            jax.ShapeDtypeStruct((NW, 16), jnp.int32),
            jax.ShapeDtypeStruct((NC, N4), jnp.float32),
        ],
        mesh=_mesh,
        scratch_types=[
            pltpu.VMEM((3 * N,), jnp.float32),
            pltpu.VMEM((N,), jnp.float32),
            pltpu.VMEM((EPW1_PAD,), jnp.int32),
            pltpu.VMEM((EPW1_PAD,), jnp.int32),
            pltpu.VMEM((16,), jnp.float32),
            pltpu.VMEM((N4,), jnp.float32),
            pltpu.VMEM((REG,), jnp.int32),
            pltpu.VMEM((REG,), jnp.int32),
            pltpu.VMEM((REG,), jnp.float32),
            pltpu.VMEM((16,), jnp.int32),
            pltpu.VMEM((N4 // NPASS // NS,), jnp.float32),
            pltpu.VMEM((NS, N4 // NPASS // NS), jnp.float32),
            pltpu.VMEM_SHARED((NS, N4 // NPASS), jnp.float32),
        ],
        compiler_params=_sc_params,
    )
    return kern(colors_flat, probas, src, dst, w0_pad)


def _sc2_half(h1x_hbm, outx_hbm, srcc_hbm, dstc_hbm, wc_hbm, counts_hbm,
              src_v, wbuf, dbuf, cntbuf, rows, acc, s):
    zero16 = jnp.zeros((16,), jnp.float32)

    @pl.loop(0, CH)
    def _zz(i):
        for t in range(8):
            rows[i, pl.ds(t * 16, 16)] = zero16

    row0 = s * RPS
    off = 0
    for blk in WB_BLOCKS:
        pltpu.sync_copy(rows.at[pl.ds(0, blk)],
                        acc.at[pl.ds(row0 + off, blk)])
        off += blk

    @pl.when(s == NS - 1)
    def _ztail():
        pltpu.sync_copy(rows.at[pl.ds(0, 16)],
                        acc.at[pl.ds(NS * RPS, 16)])

    plsc.subcore_barrier()

    for r in range(2):
        wreg = s * 2 + r
        pltpu.sync_copy(counts_hbm.at[wreg], cntbuf)
        cnt = cntbuf[pl.ds(0, 16)][0]
        nch = lax.shift_right_logical(cnt + (CH - 1), 7)
        pltpu.sync_copy(srcc_hbm.at[wreg], src_v)

        def _chunk(k, carry):
            pltpu.sync_copy(wc_hbm.at[wreg, pl.ds(k * CH, CH)], wbuf)
            pltpu.sync_copy(dstc_hbm.at[wreg, pl.ds(k * CH, CH)],
                            dbuf.at[0])
            pltpu.sync_copy(h1x_hbm.at[src_v.at[pl.ds(k * CH, CH)]], rows)

            @pl.loop(0, CH, step=16)
            def _scale(g):
                wgrp = wbuf[pl.ds(g, 16)]
                for j in range(16):
                    wj = wgrp[j]
                    for t in range(8):
                        rows[g + j, pl.ds(t * 16, 16)] = (
                            rows[g + j, pl.ds(t * 16, 16)] * wj)

            pltpu.sync_copy(rows, acc.at[dbuf.at[0]], add=True)
            return carry

        lax.fori_loop(0, nch, _chunk, jnp.int32(0))

    plsc.subcore_barrier()
    off = 0
    for blk in WB_BLOCKS:
        rr = row0 + off
        pltpu.sync_copy(acc.at[pl.ds(rr, blk)], outx_hbm.at[pl.ds(rr, blk)])
        off += blk

    @pl.when(s == NS - 1)
    def _wtail():
        pltpu.sync_copy(acc.at[pl.ds(NS * RPS, 16)],
                        outx_hbm.at[pl.ds(NS * RPS, 16)])


def _sc2_body(h1a_hbm, h1b_hbm, srcc_hbm, dstc_hbm, wc_hbm, counts_hbm,
              z2a_hbm, z2b_hbm,
              src_v, wbuf, dbuf, cntbuf, rows, acc):
    c = lax.axis_index("c")
    s = lax.axis_index("s")

    @pl.when(c == 0)
    def _():
        _sc2_half(h1a_hbm, z2a_hbm, srcc_hbm, dstc_hbm, wc_hbm, counts_hbm,
                  src_v, wbuf, dbuf, cntbuf, rows, acc, s)

    @pl.when(c == 1)
    def _():
        _sc2_half(h1b_hbm, z2b_hbm, srcc_hbm, dstc_hbm, wc_hbm, counts_hbm,
                  src_v, wbuf, dbuf, cntbuf, rows, acc, s)


def _sc2(h1a, h1b, srcc, dstc, wc, counts):
    kern = pl.kernel(
        _sc2_body,
        out_type=[
            jax.ShapeDtypeStruct((N, 128), jnp.float32),
            jax.ShapeDtypeStruct((N, 128), jnp.float32),
        ],
        mesh=_mesh,
        scratch_types=[
            pltpu.VMEM((REG,), jnp.int32),
            pltpu.VMEM((CH,), jnp.float32),
            pltpu.VMEM((1, CH), jnp.int32),
            pltpu.VMEM((16,), jnp.int32),
            pltpu.VMEM((CH, 128), jnp.float32),
            pltpu.VMEM_SHARED((N, 128), jnp.float32),
        ],
        compiler_params=_sc_params,
    )
    return kern(h1a, h1b, srcc, dstc, wc, counts)


def _g_body(feats_ref, w1b_ref, out_ref):
    out_ref[...] = jax.lax.dot_general(
        feats_ref[...], w1b_ref[...], (((1,), (0,)), ((), ())),
        preferred_element_type=jnp.float32,
        precision=jax.lax.Precision.HIGHEST)


def _tc_g(feats, w1b):
    bn = 1000
    return pl.pallas_call(
        _g_body,
        grid=(N // bn,),
        in_specs=[
            pl.BlockSpec((bn, 512), lambda i: (i, 0)),
            pl.BlockSpec((512, 256), lambda i: (0, 0)),
        ],
        out_specs=pl.BlockSpec((bn, 256), lambda i: (i, 0)),
        out_shape=jax.ShapeDtypeStruct((N, 256), jnp.float32),
    )(feats, w1b)


def _b_body(agg_ref, colors_ref, g_ref, w0_ref, b0_ref, w1a_ref,
            h1a_ref, h1b_ref):
    agg = agg_ref[0, :, 0:3] + agg_ref[1, :, 0:3]
    h0 = jax.lax.dot_general(
        colors_ref[...], w0_ref[...], (((1,), (0,)), ((), ())),
        preferred_element_type=jnp.float32,
        precision=jax.lax.Precision.HIGHEST)
    r = jnp.maximum(agg + h0 + b0_ref[...], 0.0)
    h1 = jax.lax.dot_general(
        r, w1a_ref[...], (((1,), (0,)), ((), ())),
        preferred_element_type=jnp.float32,
        precision=jax.lax.Precision.HIGHEST) + g_ref[...]
    h1a_ref[...] = h1[:, :128]
    h1b_ref[...] = h1[:, 128:]


def _tc_b(agg0, colors, g, w0, b0_row, w1a):
    bn = 1000
    agg0_r = agg0.reshape(NC, N4 // 4, 4)
    return pl.pallas_call(
        _b_body,
        grid=(N // bn,),
        in_specs=[
            pl.BlockSpec((NC, bn, 4), lambda i: (0, i, 0)),
            pl.BlockSpec((bn, 3), lambda i: (i, 0)),
            pl.BlockSpec((bn, 256), lambda i: (i, 0)),
            pl.BlockSpec((3, 3), lambda i: (0, 0)),
            pl.BlockSpec((1, 3), lambda i: (0, 0)),
            pl.BlockSpec((3, 256), lambda i: (0, 0)),
        ],
        out_specs=[
            pl.BlockSpec((bn, 128), lambda i: (i, 0)),
            pl.BlockSpec((bn, 128), lambda i: (i, 0)),
        ],
        out_shape=[
            jax.ShapeDtypeStruct((N, 128), jnp.float32),
            jax.ShapeDtypeStruct((N, 128), jnp.float32),
        ],
    )(agg0_r, colors, g, w0, b0_row, w1a)


def _c_body(z2a_ref, z2b_ref, h1a_ref, h1b_ref, aspp_ref, b1_ref, mu_ref,
            out_ref):
    agg = jnp.concatenate([z2a_ref[...], z2b_ref[...]], axis=1)
    h1 = jnp.concatenate([h1a_ref[...], h1b_ref[...]], axis=1)
    z2 = 0.5 * (agg + h1 + b1_ref[...] + aspp_ref[...])
    mu = mu_ref[...]
    dot = jax.lax.dot_general(
        z2, mu, (((1,), (1,)), ((), ())),
        preferred_element_type=jnp.float32,
        precision=jax.lax.Precision.HIGHEST)
    z2sq = jnp.sum(z2 * z2, axis=1, keepdims=True)
    musq = jnp.sum(mu * mu, axis=1)[None, :]
    d2 = jnp.maximum(z2sq + musq - 2.0 * dot, 0.0)
    f2 = 1.0 / (1.0 + d2)
    fmax = jnp.max(f2, axis=1, keepdims=True)
    ef = jnp.exp(f2 - fmax)
    out_ref[...] = ef / jnp.sum(ef, axis=1, keepdims=True)


def _tc_c(z2a, z2b, h1a, h1b, aspp, b1_row, mu):
    bn = 1000
    return pl.pallas_call(
        _c_body,
        grid=(N // bn,),
        in_specs=[
            pl.BlockSpec((bn, 128), lambda i: (i, 0)),
            pl.BlockSpec((bn, 128), lambda i: (i, 0)),
            pl.BlockSpec((bn, 128), lambda i: (i, 0)),
            pl.BlockSpec((bn, 128), lambda i: (i, 0)),
            pl.BlockSpec((bn, 256), lambda i: (i, 0)),
            pl.BlockSpec((1, 256), lambda i: (0, 0)),
            pl.BlockSpec((K, 256), lambda i: (0, 0)),
        ],
        out_specs=pl.BlockSpec((bn, K), lambda i: (i, 0)),
        out_shape=jax.ShapeDtypeStruct((N, K), jnp.float32),
    )(z2a, z2b, h1a, h1b, aspp, b1_row, mu)


def kernel(nodes_color, probas, feats_pooled, pooled_aspp_feats, edges_nn,
           W0, b0, W1, b1, mu):
    src = edges_nn[:, 0]
    dst = edges_nn[:, 1]
    colors_flat = nodes_color.T.reshape(-1)
    w0_pad = jnp.pad(W0.reshape(-1), (0, 7))

    srcc, dstc, wc, counts, agg0 = _sc1(colors_flat, probas, src, dst, w0_pad)
    g = _tc_g(feats_pooled, W1[3:])
    h1a, h1b = _tc_b(agg0, nodes_color, g, W0, b0.reshape(1, 3), W1[:3])

    z2a, z2b = _sc2(h1a, h1b, srcc, dstc, wc, counts)

    return _tc_c(z2a, z2b, h1a, h1b, pooled_aspp_feats,
                 b1.reshape(1, 256), mu)


# speedup vs baseline: 1.3280x; 1.0074x over previous
import dataclasses

import jax
import jax.numpy as jnp
from jax import lax
from jax.experimental import pallas as pl
from jax.experimental.pallas import tpu as pltpu
from jax.experimental.pallas import tpu_sc as plsc

N = 10000
E = 160000
K = 30

NC = 2
NS = 16
NW = NC * NS

EPW1 = E // NW
EPW1_PAD = 5008
REG = 5120
N4 = 40960
NPASS = 4

CH = 128

RPS = 624
WB_BLOCKS = (128, 128, 128, 128, 112)

_mesh = plsc.VectorSubcoreMesh(core_axis_name="c", subcore_axis_name="s")

_sc_params = pltpu.CompilerParams()
if "needs_layout_passes" in pltpu.CompilerParams.__dataclass_fields__:
    _sc_params = dataclasses.replace(_sc_params, needs_layout_passes=False)


def _sc1_body(colors_hbm, probas_hbm, src_hbm, dst_hbm, w0_hbm,
              srcc_out, dstc_out, wc_out, counts_out, agg0_out,
              colors_v, probas_v, src_v, dst_v, w0_v, acc_v,
              cs_v, cd_v, cw_v, cnt_v, red_a, red_in, stage):
    c = lax.axis_index("c")
    s = lax.axis_index("s")
    wid = c * NS + s

    pltpu.sync_copy(colors_hbm, colors_v)
    pltpu.sync_copy(probas_hbm, probas_v)
    pltpu.sync_copy(w0_hbm, w0_v)
    base_e = wid * EPW1
    pltpu.sync_copy(src_hbm.at[pl.ds(base_e, EPW1)], src_v.at[pl.ds(0, EPW1)])
    pltpu.sync_copy(dst_hbm.at[pl.ds(base_e, EPW1)], dst_v.at[pl.ds(0, EPW1)])

    zero16 = jnp.zeros((16,), jnp.float32)
    izero16 = jnp.zeros((16,), jnp.int32)

    @pl.loop(0, N4, step=16)
    def _zero(i):
        acc_v[pl.ds(i, 16)] = zero16

    @pl.loop(0, REG, step=16)
    def _zeroc(i):
        cs_v[pl.ds(i, 16)] = izero16
        cd_v[pl.ds(i, 16)] = izero16
        cw_v[pl.ds(i, 16)] = zero16

    nmax = jnp.full((16,), N - 1, jnp.int32)
    lane = lax.iota(jnp.int32, 16)
    half = jnp.full((16,), 0.5, jnp.float32)

    w0vec = w0_v[pl.ds(0, 16)]
    w00 = w0vec[0]
    w01 = w0vec[1]
    w02 = w0vec[2]
    w10 = w0vec[3]
    w11 = w0vec[4]
    w12 = w0vec[5]
    w20 = w0vec[6]
    w21 = w0vec[7]
    w22 = w0vec[8]

    def _edge_body(i, ptr):
        b = i * 16
        valid = (lane + b) < EPW1
        src16 = jnp.minimum(jnp.maximum(src_v[pl.ds(b, 16)], izero16), nmax)
        dst16 = jnp.minimum(jnp.maximum(dst_v[pl.ds(b, 16)], izero16), nmax)
        ps = plsc.load_gather(probas_v, [src16])
        pd = plsc.load_gather(probas_v, [dst16])
        m = ((ps >= half) & (pd >= half)) | ((ps < half) & (pd < half))
        cs0 = plsc.load_gather(colors_v, [src16])
        cd0 = plsc.load_gather(colors_v, [dst16])
        cs1 = plsc.load_gather(colors_v, [src16 + N])
        cd1 = plsc.load_gather(colors_v, [dst16 + N])
        cs2 = plsc.load_gather(colors_v, [src16 + 2 * N])
        cd2 = plsc.load_gather(colors_v, [dst16 + 2 * N])
        d0 = cs0 - cd0
        d1 = cs1 - cd1
        d2c = cs2 - cd2
        dist = d0 * d0 + d1 * d1 + d2c * d2c
        s0 = jnp.exp(dist * jnp.float32(-1.0 / 255.0))
        wv = jnp.where(m, s0, jnp.float32(0.0))
        h0a = cs0 * w00 + cs1 * w10 + cs2 * w20
        h0b = cs0 * w01 + cs1 * w11 + cs2 * w21
        h0c = cs0 * w02 + cs1 * w12 + cs2 * w22
        di = dst16 * 4
        plsc.addupdate_scatter(acc_v, [di], wv * h0a, mask=valid)
        plsc.addupdate_scatter(acc_v, [di + 1], wv * h0b, mask=valid)
        plsc.addupdate_scatter(acc_v, [di + 2], wv * h0c, mask=valid)
        keep = m & valid
        plsc.store_compressed(cs_v.at[pl.ds(ptr, 16)], src16, mask=keep)
        plsc.store_compressed(cd_v.at[pl.ds(ptr, 16)], dst16, mask=keep)
        plsc.store_compressed(cw_v.at[pl.ds(ptr, 16)], wv, mask=keep)
        return ptr + plsc.all_reduce_population_count(keep)[0]

    nkept = lax.fori_loop(0, EPW1_PAD // 16, _edge_body, jnp.int32(0))

    pltpu.sync_copy(cs_v, srcc_out.at[wid])
    pltpu.sync_copy(cd_v, dstc_out.at[wid])
    pltpu.sync_copy(cw_v, wc_out.at[wid])
    cnt_v[pl.ds(0, 16)] = jnp.where(lane == 0, nkept, 0)
    pltpu.sync_copy(cnt_v, counts_out.at[wid])

    part_sz = N4 // NPASS
    seg = part_sz // NS
    off = s * seg
    for part in range(NPASS):
        pltpu.sync_copy(acc_v.at[pl.ds(part * part_sz, part_sz)],
                        stage.at[s])
        plsc.subcore_barrier()
        pltpu.sync_copy(stage.at[:, pl.ds(off, seg)], red_in)

        @pl.loop(0, seg, step=16)
        def _add(i):
            tot = red_in[0, pl.ds(i, 16)]
            for p in range(1, NS):
                tot = tot + red_in[p, pl.ds(i, 16)]
            red_a[pl.ds(i, 16)] = tot

        pltpu.sync_copy(red_a,
                        agg0_out.at[c, pl.ds(part * part_sz + off, seg)])
        plsc.subcore_barrier()


def _sc1(colors_flat, probas, src, dst, w0_pad):
    kern = pl.kernel(
        _sc1_body,
        out_type=[
            jax.ShapeDtypeStruct((NW, REG), jnp.int32),
            jax.ShapeDtypeStruct((NW, REG), jnp.int32),
            jax.ShapeDtypeStruct((NW, REG), jnp.float32),
            jax.ShapeDtypeStruct((NW, 16), jnp.int32),
            jax.ShapeDtypeStruct((NC, N4), jnp.float32),
        ],
        mesh=_mesh,
        scratch_types=[
            pltpu.VMEM((3 * N,), jnp.float32),
            pltpu.VMEM((N,), jnp.float32),
            pltpu.VMEM((EPW1_PAD,), jnp.int32),
            pltpu.VMEM((EPW1_PAD,), jnp.int32),
            pltpu.VMEM((16,), jnp.float32),
            pltpu.VMEM((N4,), jnp.float32),
            pltpu.VMEM((REG,), jnp.int32),
            pltpu.VMEM((REG,), jnp.int32),
            pltpu.VMEM((REG,), jnp.float32),
            pltpu.VMEM((16,), jnp.int32),
            pltpu.VMEM((N4 // NPASS // NS,), jnp.float32),
            pltpu.VMEM((NS, N4 // NPASS // NS), jnp.float32),
            pltpu.VMEM_SHARED((NS, N4 // NPASS), jnp.float32),
        ],
        compiler_params=_sc_params,
    )
    return kern(colors_flat, probas, src, dst, w0_pad)


def _sc2_half(h1x_hbm, outx_hbm, srcc_hbm, dstc_hbm, wc_hbm, counts_hbm,
              src_v, wbuf, dbuf, cntbuf, rows, acc, s):
    zero16 = jnp.zeros((16,), jnp.float32)

    @pl.loop(0, CH)
    def _zz(i):
        for t in range(8):
            rows[i, pl.ds(t * 16, 16)] = zero16

    row0 = s * RPS
    off = 0
    for blk in WB_BLOCKS:
        pltpu.sync_copy(rows.at[pl.ds(0, blk)],
                        acc.at[pl.ds(row0 + off, blk)])
        off += blk

    @pl.when(s == NS - 1)
    def _ztail():
        pltpu.sync_copy(rows.at[pl.ds(0, 16)],
                        acc.at[pl.ds(NS * RPS, 16)])

    plsc.subcore_barrier()

    for r in range(2):
        wreg = s * 2 + r
        pltpu.sync_copy(counts_hbm.at[wreg], cntbuf)
        cnt = cntbuf[pl.ds(0, 16)][0]
        nch = lax.shift_right_logical(cnt + (CH - 1), 7)
        pltpu.sync_copy(srcc_hbm.at[wreg], src_v)

        def _chunk(k, carry):
            pltpu.sync_copy(wc_hbm.at[wreg, pl.ds(k * CH, CH)], wbuf)
            pltpu.sync_copy(dstc_hbm.at[wreg, pl.ds(k * CH, CH)],
                            dbuf.at[0])
            pltpu.sync_copy(h1x_hbm.at[src_v.at[pl.ds(k * CH, CH)]], rows)

            @pl.loop(0, CH, step=16)
            def _scale(g):
                wgrp = wbuf[pl.ds(g, 16)]
                for j in range(16):
                    wj = wgrp[j]
                    for t in range(8):
                        rows[g + j, pl.ds(t * 16, 16)] = (
                            rows[g + j, pl.ds(t * 16, 16)] * wj)

            pltpu.sync_copy(rows, acc.at[dbuf.at[0]], add=True)
            return carry

        lax.fori_loop(0, nch, _chunk, jnp.int32(0))

    plsc.subcore_barrier()
    off = 0
    for blk in WB_BLOCKS:
        rr = row0 + off
        pltpu.sync_copy(acc.at[pl.ds(rr, blk)], outx_hbm.at[pl.ds(rr, blk)])
        off += blk

    @pl.when(s == NS - 1)
    def _wtail():
        pltpu.sync_copy(acc.at[pl.ds(NS * RPS, 16)],
                        outx_hbm.at[pl.ds(NS * RPS, 16)])


def _sc2_body(h1a_hbm, h1b_hbm, srcc_hbm, dstc_hbm, wc_hbm, counts_hbm,
              z2a_hbm, z2b_hbm,
              src_v, wbuf, dbuf, cntbuf, rows, acc):
    c = lax.axis_index("c")
    s = lax.axis_index("s")

    @pl.when(c == 0)
    def _():
        _sc2_half(h1a_hbm, z2a_hbm, srcc_hbm, dstc_hbm, wc_hbm, counts_hbm,
                  src_v, wbuf, dbuf, cntbuf, rows, acc, s)

    @pl.when(c == 1)
    def _():
        _sc2_half(h1b_hbm, z2b_hbm, srcc_hbm, dstc_hbm, wc_hbm, counts_hbm,
                  src_v, wbuf, dbuf, cntbuf, rows, acc, s)


def _sc2(h1a, h1b, srcc, dstc, wc, counts):
    kern = pl.kernel(
        _sc2_body,
        out_type=[
            jax.ShapeDtypeStruct((N, 128), jnp.float32),
            jax.ShapeDtypeStruct((N, 128), jnp.float32),
        ],
        mesh=_mesh,
        scratch_types=[
            pltpu.VMEM((REG,), jnp.int32),
            pltpu.VMEM((CH,), jnp.float32),
            pltpu.VMEM((1, CH), jnp.int32),
            pltpu.VMEM((16,), jnp.int32),
            pltpu.VMEM((CH, 128), jnp.float32),
            pltpu.VMEM_SHARED((N, 128), jnp.float32),
        ],
        compiler_params=_sc_params,
    )
    return kern(h1a, h1b, srcc, dstc, wc, counts)


def _g_body(feats_ref, w1b_ref, out_ref):
    out_ref[...] = jax.lax.dot_general(
        feats_ref[...], w1b_ref[...], (((1,), (0,)), ((), ())),
        preferred_element_type=jnp.float32,
        precision=jax.lax.Precision.HIGHEST)


def _tc_g(feats, w1b):
    bn = 1000
    return pl.pallas_call(
        _g_body,
        grid=(N // bn,),
        in_specs=[
            pl.BlockSpec((bn, 512), lambda i: (i, 0)),
            pl.BlockSpec((512, 256), lambda i: (0, 0)),
        ],
        out_specs=pl.BlockSpec((bn, 256), lambda i: (i, 0)),
        out_shape=jax.ShapeDtypeStruct((N, 256), jnp.float32),
    )(feats, w1b)


def _b_body(agg_ref, colors_ref, g_ref, w0_ref, b0_ref, w1a_ref,
            h1a_ref, h1b_ref):
    agg = agg_ref[0, :, 0:3] + agg_ref[1, :, 0:3]
    h0 = jax.lax.dot_general(
        colors_ref[...], w0_ref[...], (((1,), (0,)), ((), ())),
        preferred_element_type=jnp.float32,
        precision=jax.lax.Precision.HIGHEST)
    r = jnp.maximum(agg + h0 + b0_ref[...], 0.0)
    h1 = jax.lax.dot_general(
        r, w1a_ref[...], (((1,), (0,)), ((), ())),
        preferred_element_type=jnp.float32,
        precision=jax.lax.Precision.HIGHEST) + g_ref[...]
    h1a_ref[...] = h1[:, :128]
    h1b_ref[...] = h1[:, 128:]


def _tc_b(agg0, colors, g, w0, b0_row, w1a):
    bn = 1000
    agg0_r = agg0.reshape(NC, N4 // 4, 4)
    return pl.pallas_call(
        _b_body,
        grid=(N // bn,),
        in_specs=[
            pl.BlockSpec((NC, bn, 4), lambda i: (0, i, 0)),
            pl.BlockSpec((bn, 3), lambda i: (i, 0)),
            pl.BlockSpec((bn, 256), lambda i: (i, 0)),
            pl.BlockSpec((3, 3), lambda i: (0, 0)),
            pl.BlockSpec((1, 3), lambda i: (0, 0)),
            pl.BlockSpec((3, 256), lambda i: (0, 0)),
        ],
        out_specs=[
            pl.BlockSpec((bn, 128), lambda i: (i, 0)),
            pl.BlockSpec((bn, 128), lambda i: (i, 0)),
        ],
        out_shape=[
            jax.ShapeDtypeStruct((N, 128), jnp.float32),
            jax.ShapeDtypeStruct((N, 128), jnp.float32),
        ],
    )(agg0_r, colors, g, w0, b0_row, w1a)


def _c_body(z2a_ref, z2b_ref, h1a_ref, h1b_ref, aspp_ref, b1_ref, mu_ref,
            out_ref):
    agg = jnp.concatenate([z2a_ref[...], z2b_ref[...]], axis=1)
    h1 = jnp.concatenate([h1a_ref[...], h1b_ref[...]], axis=1)
    z2 = 0.5 * (agg + h1 + b1_ref[...] + aspp_ref[...])
    mu = mu_ref[...]
    dot = jax.lax.dot_general(
        z2, mu, (((1,), (1,)), ((), ())),
        preferred_element_type=jnp.float32,
        precision=jax.lax.Precision.HIGHEST)
    z2sq = jnp.sum(z2 * z2, axis=1, keepdims=True)
    musq = jnp.sum(mu * mu, axis=1)[None, :]
    d2 = jnp.maximum(z2sq + musq - 2.0 * dot, 0.0)
    f2 = 1.0 / (1.0 + d2)
    fmax = jnp.max(f2, axis=1, keepdims=True)
    ef = jnp.exp(f2 - fmax)
    out_ref[...] = ef / jnp.sum(ef, axis=1, keepdims=True)


def _tc_c(z2a, z2b, h1a, h1b, aspp, b1_row, mu):
    bn = 1000
    return pl.pallas_call(
        _c_body,
        grid=(N // bn,),
        in_specs=[
            pl.BlockSpec((bn, 128), lambda i: (i, 0)),
            pl.BlockSpec((bn, 128), lambda i: (i, 0)),
            pl.BlockSpec((bn, 128), lambda i: (i, 0)),
            pl.BlockSpec((bn, 128), lambda i: (i, 0)),
            pl.BlockSpec((bn, 256), lambda i: (i, 0)),
            pl.BlockSpec((1, 256), lambda i: (0, 0)),
            pl.BlockSpec((K, 256), lambda i: (0, 0)),
        ],
        out_specs=pl.BlockSpec((bn, K), lambda i: (i, 0)),
        out_shape=jax.ShapeDtypeStruct((N, K), jnp.float32),
    )(z2a, z2b, h1a, h1b, aspp, b1_row, mu)


def kernel(nodes_color, probas, feats_pooled, pooled_aspp_feats, edges_nn,
           W0, b0, W1, b1, mu):
    src = edges_nn[:, 0]
    dst = edges_nn[:, 1]
    colors_flat = nodes_color.T.reshape(-1)
    w0_pad = jnp.pad(W0.reshape(-1), (0, 7))

    srcc, dstc, wc, counts, agg0 = _sc1(colors_flat, probas, src, dst, w0_pad)
    g = _tc_g(feats_pooled, W1[3:])
    h1a, h1b = _tc_b(agg0, nodes_color, g, W0, b0.reshape(1, 3), W1[:3])

    z2a, z2b = _sc2(h1a, h1b, srcc, dstc, wc, counts)

    return _tc_c(z2a, z2b, h1a, h1b, pooled_aspp_feats,
                 b1.reshape(1, 256), mu)


# speedup vs baseline: 1.4456x; 1.0885x over previous
import dataclasses

import jax
import jax.numpy as jnp
from jax import lax
from jax.experimental import pallas as pl
from jax.experimental.pallas import tpu as pltpu
from jax.experimental.pallas import tpu_sc as plsc

N = 10000
E = 160000
K = 30

NC = 2
NS = 16
NW = NC * NS

EPW1 = E // NW
EPW1_PAD = 5008
REG = 5120
N4 = 40960
NPASS = 4

CH = 128

RPS = 624
WB_BLOCKS = (128, 128, 128, 128, 112)

_mesh = plsc.VectorSubcoreMesh(core_axis_name="c", subcore_axis_name="s")

_sc_params = pltpu.CompilerParams()
if "needs_layout_passes" in pltpu.CompilerParams.__dataclass_fields__:
    _sc_params = dataclasses.replace(_sc_params, needs_layout_passes=False)


def _sc1_body(colors_hbm, probas_hbm, src_hbm, dst_hbm, w0_hbm,
              srcc_out, dstc_out, wc_out, counts_out, agg0_out,
              colors_v, probas_v, src_v, dst_v, w0_v, acc_v,
              cs_v, cd_v, cw_v, cnt_v, red_a, red_in, stage):
    c = lax.axis_index("c")
    s = lax.axis_index("s")
    wid = c * NS + s

    pltpu.sync_copy(colors_hbm, colors_v)
    pltpu.sync_copy(probas_hbm, probas_v)
    pltpu.sync_copy(w0_hbm, w0_v)
    base_e = wid * EPW1
    pltpu.sync_copy(src_hbm.at[pl.ds(base_e, EPW1)], src_v.at[pl.ds(0, EPW1)])
    pltpu.sync_copy(dst_hbm.at[pl.ds(base_e, EPW1)], dst_v.at[pl.ds(0, EPW1)])

    zero16 = jnp.zeros((16,), jnp.float32)
    izero16 = jnp.zeros((16,), jnp.int32)

    @pl.loop(0, N4, step=16)
    def _zero(i):
        acc_v[pl.ds(i, 16)] = zero16

    @pl.loop(0, REG, step=16)
    def _zeroc(i):
        cs_v[pl.ds(i, 16)] = izero16
        cd_v[pl.ds(i, 16)] = izero16
        cw_v[pl.ds(i, 16)] = zero16

    nmax = jnp.full((16,), N - 1, jnp.int32)
    lane = lax.iota(jnp.int32, 16)
    half = jnp.full((16,), 0.5, jnp.float32)

    w0vec = w0_v[pl.ds(0, 16)]
    w00 = w0vec[0]
    w01 = w0vec[1]
    w02 = w0vec[2]
    w10 = w0vec[3]
    w11 = w0vec[4]
    w12 = w0vec[5]
    w20 = w0vec[6]
    w21 = w0vec[7]
    w22 = w0vec[8]

    def _edge_body(i, ptr):
        b = i * 16
        valid = (lane + b) < EPW1
        src16 = jnp.minimum(jnp.maximum(src_v[pl.ds(b, 16)], izero16), nmax)
        dst16 = jnp.minimum(jnp.maximum(dst_v[pl.ds(b, 16)], izero16), nmax)
        ps = plsc.load_gather(probas_v, [src16])
        pd = plsc.load_gather(probas_v, [dst16])
        m = ((ps >= half) & (pd >= half)) | ((ps < half) & (pd < half))
        cs0 = plsc.load_gather(colors_v, [src16])
        cd0 = plsc.load_gather(colors_v, [dst16])
        cs1 = plsc.load_gather(colors_v, [src16 + N])
        cd1 = plsc.load_gather(colors_v, [dst16 + N])
        cs2 = plsc.load_gather(colors_v, [src16 + 2 * N])
        cd2 = plsc.load_gather(colors_v, [dst16 + 2 * N])
        d0 = cs0 - cd0
        d1 = cs1 - cd1
        d2c = cs2 - cd2
        dist = d0 * d0 + d1 * d1 + d2c * d2c
        s0 = jnp.exp(dist * jnp.float32(-1.0 / 255.0))
        wv = jnp.where(m, s0, jnp.float32(0.0))
        h0a = cs0 * w00 + cs1 * w10 + cs2 * w20
        h0b = cs0 * w01 + cs1 * w11 + cs2 * w21
        h0c = cs0 * w02 + cs1 * w12 + cs2 * w22
        di = dst16 * 4
        plsc.addupdate_scatter(acc_v, [di], wv * h0a, mask=valid)
        plsc.addupdate_scatter(acc_v, [di + 1], wv * h0b, mask=valid)
        plsc.addupdate_scatter(acc_v, [di + 2], wv * h0c, mask=valid)
        keep = m & valid
        plsc.store_compressed(cs_v.at[pl.ds(ptr, 16)], src16, mask=keep)
        plsc.store_compressed(cd_v.at[pl.ds(ptr, 16)], dst16, mask=keep)
        plsc.store_compressed(cw_v.at[pl.ds(ptr, 16)], wv, mask=keep)
        return ptr + plsc.all_reduce_population_count(keep)[0]

    nkept = lax.fori_loop(0, EPW1_PAD // 16, _edge_body, jnp.int32(0))

    pltpu.sync_copy(cs_v, srcc_out.at[wid])
    pltpu.sync_copy(cd_v, dstc_out.at[wid])
    pltpu.sync_copy(cw_v, wc_out.at[wid])
    cnt_v[pl.ds(0, 16)] = jnp.where(lane == 0, nkept, 0)
    pltpu.sync_copy(cnt_v, counts_out.at[wid])

    part_sz = N4 // NPASS
    seg = part_sz // NS
    off = s * seg
    for part in range(NPASS):
        pltpu.sync_copy(acc_v.at[pl.ds(part * part_sz, part_sz)],
                        stage.at[s])
        plsc.subcore_barrier()
        pltpu.sync_copy(stage.at[:, pl.ds(off, seg)], red_in)

        @pl.loop(0, seg, step=16)
        def _add(i):
            tot = red_in[0, pl.ds(i, 16)]
            for p in range(1, NS):
                tot = tot + red_in[p, pl.ds(i, 16)]
            red_a[pl.ds(i, 16)] = tot

        pltpu.sync_copy(red_a,
                        agg0_out.at[c, pl.ds(part * part_sz + off, seg)])
        plsc.subcore_barrier()


def _sc1(colors_flat, probas, src, dst, w0_pad):
    kern = pl.kernel(
        _sc1_body,
        out_type=[
            jax.ShapeDtypeStruct((NW, REG), jnp.int32),
            jax.ShapeDtypeStruct((NW, REG), jnp.int32),
            jax.ShapeDtypeStruct((NW, REG), jnp.float32),
            jax.ShapeDtypeStruct((NW, 16), jnp.int32),
            jax.ShapeDtypeStruct((NC, N4), jnp.float32),
        ],
        mesh=_mesh,
        scratch_types=[
            pltpu.VMEM((3 * N,), jnp.float32),
            pltpu.VMEM((N,), jnp.float32),
            pltpu.VMEM((EPW1_PAD,), jnp.int32),
            pltpu.VMEM((EPW1_PAD,), jnp.int32),
            pltpu.VMEM((16,), jnp.float32),
            pltpu.VMEM((N4,), jnp.float32),
            pltpu.VMEM((REG,), jnp.int32),
            pltpu.VMEM((REG,), jnp.int32),
            pltpu.VMEM((REG,), jnp.float32),
            pltpu.VMEM((16,), jnp.int32),
            pltpu.VMEM((N4 // NPASS // NS,), jnp.float32),
            pltpu.VMEM((NS, N4 // NPASS // NS), jnp.float32),
            pltpu.VMEM_SHARED((NS, N4 // NPASS), jnp.float32),
        ],
        compiler_params=_sc_params,
    )
    return kern(colors_flat, probas, src, dst, w0_pad)


def _sc2_half(h1x_hbm, outx_hbm, srcc_hbm, dstc_hbm, wc_hbm, counts_hbm,
              src_v, wbuf, dbuf, cntbuf, rows, acc, gsem, msem, s):
    zero16 = jnp.zeros((16,), jnp.float32)

    @pl.loop(0, CH)
    def _zz(i):
        for t in range(8):
            rows[i, pl.ds(t * 16, 16)] = zero16

    row0 = s * RPS
    off = 0
    for blk in WB_BLOCKS:
        pltpu.sync_copy(rows.at[pl.ds(0, blk)],
                        acc.at[pl.ds(row0 + off, blk)])
        off += blk

    @pl.when(s == NS - 1)
    def _ztail():
        pltpu.sync_copy(rows.at[pl.ds(0, 16)],
                        acc.at[pl.ds(NS * RPS, 16)])

    plsc.subcore_barrier()

    for r in range(2):
        wreg = s * 2 + r
        pltpu.sync_copy(counts_hbm.at[wreg], cntbuf)
        cnt = cntbuf[pl.ds(0, 16)][0]
        nch = lax.shift_right_logical(cnt + (CH - 1), 7)
        pltpu.sync_copy(srcc_hbm.at[wreg], src_v)
        pltpu.sync_copy(wc_hbm.at[wreg], wbuf)

        def _chunk(k, carry):
            pltpu.make_async_copy(dstc_hbm.at[wreg, pl.ds(k * CH, CH)],
                                  dbuf.at[0], msem).start()
            pltpu.make_async_copy(h1x_hbm.at[src_v.at[pl.ds(k * CH, CH)]],
                                  rows, gsem).start()
            pltpu.make_async_copy(dstc_hbm.at[wreg, pl.ds(0, CH)],
                                  dbuf.at[0], msem).wait()
            pltpu.make_async_copy(h1x_hbm.at[src_v.at[pl.ds(0, CH)]],
                                  rows, gsem).wait()

            @pl.loop(0, CH, step=16)
            def _scale(g):
                wgrp = wbuf[pl.ds(k * CH + g, 16)]
                for j in range(16):
                    wj = wgrp[j]
                    for t in range(8):
                        rows[g + j, pl.ds(t * 16, 16)] = (
                            rows[g + j, pl.ds(t * 16, 16)] * wj)

            pltpu.sync_copy(rows, acc.at[dbuf.at[0]], add=True)
            return carry

        lax.fori_loop(0, nch, _chunk, jnp.int32(0))

    plsc.subcore_barrier()
    off = 0
    for blk in WB_BLOCKS:
        rr = row0 + off
        pltpu.sync_copy(acc.at[pl.ds(rr, blk)], outx_hbm.at[pl.ds(rr, blk)])
        off += blk

    @pl.when(s == NS - 1)
    def _wtail():
        pltpu.sync_copy(acc.at[pl.ds(NS * RPS, 16)],
                        outx_hbm.at[pl.ds(NS * RPS, 16)])


def _sc2_body(h1a_hbm, h1b_hbm, srcc_hbm, dstc_hbm, wc_hbm, counts_hbm,
              z2a_hbm, z2b_hbm,
              src_v, wbuf, dbuf, cntbuf, rows, acc, gsem, msem):
    c = lax.axis_index("c")
    s = lax.axis_index("s")

    @pl.when(c == 0)
    def _():
        _sc2_half(h1a_hbm, z2a_hbm, srcc_hbm, dstc_hbm, wc_hbm, counts_hbm,
                  src_v, wbuf, dbuf, cntbuf, rows, acc, gsem, msem, s)

    @pl.when(c == 1)
    def _():
        _sc2_half(h1b_hbm, z2b_hbm, srcc_hbm, dstc_hbm, wc_hbm, counts_hbm,
                  src_v, wbuf, dbuf, cntbuf, rows, acc, gsem, msem, s)


def _sc2(h1a, h1b, srcc, dstc, wc, counts):
    kern = pl.kernel(
        _sc2_body,
        out_type=[
            jax.ShapeDtypeStruct((N, 128), jnp.float32),
            jax.ShapeDtypeStruct((N, 128), jnp.float32),
        ],
        mesh=_mesh,
        scratch_types=[
            pltpu.VMEM((REG,), jnp.int32),
            pltpu.VMEM((REG,), jnp.float32),
            pltpu.VMEM((1, CH), jnp.int32),
            pltpu.VMEM((16,), jnp.int32),
            pltpu.VMEM((CH, 128), jnp.float32),
            pltpu.VMEM_SHARED((N, 128), jnp.float32),
            pltpu.SemaphoreType.DMA,
            pltpu.SemaphoreType.DMA,
        ],
        compiler_params=_sc_params,
    )
    return kern(h1a, h1b, srcc, dstc, wc, counts)


def _g_body(feats_ref, w1b_ref, out_ref):
    out_ref[...] = jax.lax.dot_general(
        feats_ref[...], w1b_ref[...], (((1,), (0,)), ((), ())),
        preferred_element_type=jnp.float32,
        precision=jax.lax.Precision.HIGHEST)


def _tc_g(feats, w1b):
    bn = 1000
    return pl.pallas_call(
        _g_body,
        grid=(N // bn,),
        in_specs=[
            pl.BlockSpec((bn, 512), lambda i: (i, 0)),
            pl.BlockSpec((512, 256), lambda i: (0, 0)),
        ],
        out_specs=pl.BlockSpec((bn, 256), lambda i: (i, 0)),
        out_shape=jax.ShapeDtypeStruct((N, 256), jnp.float32),
    )(feats, w1b)


def _b_body(agg_ref, colors_ref, g_ref, w0_ref, b0_ref, w1a_ref,
            h1a_ref, h1b_ref):
    agg = agg_ref[0, :, 0:3] + agg_ref[1, :, 0:3]
    h0 = jax.lax.dot_general(
        colors_ref[...], w0_ref[...], (((1,), (0,)), ((), ())),
        preferred_element_type=jnp.float32,
        precision=jax.lax.Precision.HIGHEST)
    r = jnp.maximum(agg + h0 + b0_ref[...], 0.0)
    h1 = jax.lax.dot_general(
        r, w1a_ref[...], (((1,), (0,)), ((), ())),
        preferred_element_type=jnp.float32,
        precision=jax.lax.Precision.HIGHEST) + g_ref[...]
    h1a_ref[...] = h1[:, :128]
    h1b_ref[...] = h1[:, 128:]


def _tc_b(agg0, colors, g, w0, b0_row, w1a):
    bn = 1000
    agg0_r = agg0.reshape(NC, N4 // 4, 4)
    return pl.pallas_call(
        _b_body,
        grid=(N // bn,),
        in_specs=[
            pl.BlockSpec((NC, bn, 4), lambda i: (0, i, 0)),
            pl.BlockSpec((bn, 3), lambda i: (i, 0)),
            pl.BlockSpec((bn, 256), lambda i: (i, 0)),
            pl.BlockSpec((3, 3), lambda i: (0, 0)),
            pl.BlockSpec((1, 3), lambda i: (0, 0)),
            pl.BlockSpec((3, 256), lambda i: (0, 0)),
        ],
        out_specs=[
            pl.BlockSpec((bn, 128), lambda i: (i, 0)),
            pl.BlockSpec((bn, 128), lambda i: (i, 0)),
        ],
        out_shape=[
            jax.ShapeDtypeStruct((N, 128), jnp.float32),
            jax.ShapeDtypeStruct((N, 128), jnp.float32),
        ],
    )(agg0_r, colors, g, w0, b0_row, w1a)


def _c_body(z2a_ref, z2b_ref, h1a_ref, h1b_ref, aspp_ref, b1_ref, mu_ref,
            out_ref):
    agg = jnp.concatenate([z2a_ref[...], z2b_ref[...]], axis=1)
    h1 = jnp.concatenate([h1a_ref[...], h1b_ref[...]], axis=1)
    z2 = 0.5 * (agg + h1 + b1_ref[...] + aspp_ref[...])
    mu = mu_ref[...]
    dot = jax.lax.dot_general(
        z2, mu, (((1,), (1,)), ((), ())),
        preferred_element_type=jnp.float32,
        precision=jax.lax.Precision.HIGHEST)
    z2sq = jnp.sum(z2 * z2, axis=1, keepdims=True)
    musq = jnp.sum(mu * mu, axis=1)[None, :]
    d2 = jnp.maximum(z2sq + musq - 2.0 * dot, 0.0)
    f2 = 1.0 / (1.0 + d2)
    fmax = jnp.max(f2, axis=1, keepdims=True)
    ef = jnp.exp(f2 - fmax)
    out_ref[...] = ef / jnp.sum(ef, axis=1, keepdims=True)


def _tc_c(z2a, z2b, h1a, h1b, aspp, b1_row, mu):
    bn = 1000
    return pl.pallas_call(
        _c_body,
        grid=(N // bn,),
        in_specs=[
            pl.BlockSpec((bn, 128), lambda i: (i, 0)),
            pl.BlockSpec((bn, 128), lambda i: (i, 0)),
            pl.BlockSpec((bn, 128), lambda i: (i, 0)),
            pl.BlockSpec((bn, 128), lambda i: (i, 0)),
            pl.BlockSpec((bn, 256), lambda i: (i, 0)),
            pl.BlockSpec((1, 256), lambda i: (0, 0)),
            pl.BlockSpec((K, 256), lambda i: (0, 0)),
        ],
        out_specs=pl.BlockSpec((bn, K), lambda i: (i, 0)),
        out_shape=jax.ShapeDtypeStruct((N, K), jnp.float32),
    )(z2a, z2b, h1a, h1b, aspp, b1_row, mu)


def kernel(nodes_color, probas, feats_pooled, pooled_aspp_feats, edges_nn,
           W0, b0, W1, b1, mu):
    src = edges_nn[:, 0]
    dst = edges_nn[:, 1]
    colors_flat = nodes_color.T.reshape(-1)
    w0_pad = jnp.pad(W0.reshape(-1), (0, 7))

    srcc, dstc, wc, counts, agg0 = _sc1(colors_flat, probas, src, dst, w0_pad)
    g = _tc_g(feats_pooled, W1[3:])
    h1a, h1b = _tc_b(agg0, nodes_color, g, W0, b0.reshape(1, 3), W1[:3])

    z2a, z2b = _sc2(h1a, h1b, srcc, dstc, wc, counts)

    return _tc_c(z2a, z2b, h1a, h1b, pooled_aspp_feats,
                 b1.reshape(1, 256), mu)


# speedup vs baseline: 1.5445x; 1.0684x over previous
import dataclasses

import jax
import jax.numpy as jnp
from jax import lax
from jax.experimental import pallas as pl
from jax.experimental.pallas import tpu as pltpu
from jax.experimental.pallas import tpu_sc as plsc

N = 10000
E = 160000
K = 30

NC = 2
NS = 16
NW = NC * NS

EPW1 = E // NW
EPW1_PAD = 5008
REG = 5120
N4 = 40960
NPASS = 4

CH = 128

RPS = 624
WB_BLOCKS = (128, 128, 128, 128, 112)

_mesh = plsc.VectorSubcoreMesh(core_axis_name="c", subcore_axis_name="s")

_sc_params = pltpu.CompilerParams()
if "needs_layout_passes" in pltpu.CompilerParams.__dataclass_fields__:
    _sc_params = dataclasses.replace(_sc_params, needs_layout_passes=False)


def _sc1_body(colors_hbm, probas_hbm, src_hbm, dst_hbm, w0_hbm,
              srcc_out, dstc_out, wc_out, counts_out, agg0_out,
              colors_v, probas_v, src_v, dst_v, w0_v, acc_v,
              cs_v, cd_v, cw_v, cnt_v, red_a, red_in, stage):
    c = lax.axis_index("c")
    s = lax.axis_index("s")
    wid = c * NS + s

    pltpu.sync_copy(colors_hbm, colors_v)
    pltpu.sync_copy(probas_hbm, probas_v)
    pltpu.sync_copy(w0_hbm, w0_v)
    base_e = wid * EPW1
    pltpu.sync_copy(src_hbm.at[pl.ds(base_e, EPW1)], src_v.at[pl.ds(0, EPW1)])
    pltpu.sync_copy(dst_hbm.at[pl.ds(base_e, EPW1)], dst_v.at[pl.ds(0, EPW1)])

    zero16 = jnp.zeros((16,), jnp.float32)
    izero16 = jnp.zeros((16,), jnp.int32)

    @pl.loop(0, N4, step=16)
    def _zero(i):
        acc_v[pl.ds(i, 16)] = zero16

    @pl.loop(0, REG, step=16)
    def _zeroc(i):
        cs_v[pl.ds(i, 16)] = izero16
        cd_v[pl.ds(i, 16)] = izero16
        cw_v[pl.ds(i, 16)] = zero16

    nmax = jnp.full((16,), N - 1, jnp.int32)
    lane = lax.iota(jnp.int32, 16)
    half = jnp.full((16,), 0.5, jnp.float32)

    w0vec = w0_v[pl.ds(0, 16)]
    w00 = w0vec[0]
    w01 = w0vec[1]
    w02 = w0vec[2]
    w10 = w0vec[3]
    w11 = w0vec[4]
    w12 = w0vec[5]
    w20 = w0vec[6]
    w21 = w0vec[7]
    w22 = w0vec[8]

    def _edge_body(i, ptr):
        b = i * 16
        valid = (lane + b) < EPW1
        src16 = jnp.minimum(jnp.maximum(src_v[pl.ds(b, 16)], izero16), nmax)
        dst16 = jnp.minimum(jnp.maximum(dst_v[pl.ds(b, 16)], izero16), nmax)
        ps = plsc.load_gather(probas_v, [src16])
        pd = plsc.load_gather(probas_v, [dst16])
        m = ((ps >= half) & (pd >= half)) | ((ps < half) & (pd < half))
        cs0 = plsc.load_gather(colors_v, [src16])
        cd0 = plsc.load_gather(colors_v, [dst16])
        cs1 = plsc.load_gather(colors_v, [src16 + N])
        cd1 = plsc.load_gather(colors_v, [dst16 + N])
        cs2 = plsc.load_gather(colors_v, [src16 + 2 * N])
        cd2 = plsc.load_gather(colors_v, [dst16 + 2 * N])
        d0 = cs0 - cd0
        d1 = cs1 - cd1
        d2c = cs2 - cd2
        dist = d0 * d0 + d1 * d1 + d2c * d2c
        s0 = jnp.exp(dist * jnp.float32(-1.0 / 255.0))
        wv = jnp.where(m, s0, jnp.float32(0.0))
        h0a = cs0 * w00 + cs1 * w10 + cs2 * w20
        h0b = cs0 * w01 + cs1 * w11 + cs2 * w21
        h0c = cs0 * w02 + cs1 * w12 + cs2 * w22
        di = dst16 * 4
        plsc.addupdate_scatter(acc_v, [di], wv * h0a, mask=valid)
        plsc.addupdate_scatter(acc_v, [di + 1], wv * h0b, mask=valid)
        plsc.addupdate_scatter(acc_v, [di + 2], wv * h0c, mask=valid)
        keep = m & valid
        plsc.store_compressed(cs_v.at[pl.ds(ptr, 16)], src16, mask=keep)
        plsc.store_compressed(cd_v.at[pl.ds(ptr, 16)], dst16, mask=keep)
        plsc.store_compressed(cw_v.at[pl.ds(ptr, 16)], wv, mask=keep)
        return ptr + plsc.all_reduce_population_count(keep)[0]

    nkept = lax.fori_loop(0, EPW1_PAD // 16, _edge_body, jnp.int32(0))

    pltpu.sync_copy(cs_v, srcc_out.at[wid])
    pltpu.sync_copy(cd_v, dstc_out.at[wid])
    pltpu.sync_copy(cw_v, wc_out.at[wid])
    cnt_v[pl.ds(0, 16)] = jnp.where(lane == 0, nkept, 0)
    pltpu.sync_copy(cnt_v, counts_out.at[wid])

    part_sz = N4 // NPASS
    seg = part_sz // NS
    off = s * seg
    for part in range(NPASS):
        pltpu.sync_copy(acc_v.at[pl.ds(part * part_sz, part_sz)],
                        stage.at[s])
        plsc.subcore_barrier()
        pltpu.sync_copy(stage.at[:, pl.ds(off, seg)], red_in)

        @pl.loop(0, seg, step=16)
        def _add(i):
            tot = red_in[0, pl.ds(i, 16)]
            for p in range(1, NS):
                tot = tot + red_in[p, pl.ds(i, 16)]
            red_a[pl.ds(i, 16)] = tot

        pltpu.sync_copy(red_a,
                        agg0_out.at[c, pl.ds(part * part_sz + off, seg)])
        plsc.subcore_barrier()


def _sc1(colors_flat, probas, src, dst, w0_pad):
    kern = pl.kernel(
        _sc1_body,
        out_type=[
            jax.ShapeDtypeStruct((NW, REG), jnp.int32),
            jax.ShapeDtypeStruct((NW, REG), jnp.int32),
            jax.ShapeDtypeStruct((NW, REG), jnp.float32),
            jax.ShapeDtypeStruct((NW, 16), jnp.int32),
            jax.ShapeDtypeStruct((NC, N4), jnp.float32),
        ],
        mesh=_mesh,
        scratch_types=[
            pltpu.VMEM((3 * N,), jnp.float32),
            pltpu.VMEM((N,), jnp.float32),
            pltpu.VMEM((EPW1_PAD,), jnp.int32),
            pltpu.VMEM((EPW1_PAD,), jnp.int32),
            pltpu.VMEM((16,), jnp.float32),
            pltpu.VMEM((N4,), jnp.float32),
            pltpu.VMEM((REG,), jnp.int32),
            pltpu.VMEM((REG,), jnp.int32),
            pltpu.VMEM((REG,), jnp.float32),
            pltpu.VMEM((16,), jnp.int32),
            pltpu.VMEM((N4 // NPASS // NS,), jnp.float32),
            pltpu.VMEM((NS, N4 // NPASS // NS), jnp.float32),
            pltpu.VMEM_SHARED((NS, N4 // NPASS), jnp.float32),
        ],
        compiler_params=_sc_params,
    )
    return kern(colors_flat, probas, src, dst, w0_pad)


def _sc2_half(h1x_hbm, outx_hbm, srcc_hbm, dstc_hbm, wc_hbm, counts_hbm,
              src_v, wbuf, dbuf, cntbuf, rows, rows2, acc, gsem, msem, s):
    zero16 = jnp.zeros((16,), jnp.float32)

    @pl.loop(0, CH)
    def _zz(i):
        for t in range(8):
            rows[i, pl.ds(t * 16, 16)] = zero16

    row0 = s * RPS
    off = 0
    for blk in WB_BLOCKS:
        pltpu.sync_copy(rows.at[pl.ds(0, blk)],
                        acc.at[pl.ds(row0 + off, blk)])
        off += blk

    @pl.when(s == NS - 1)
    def _ztail():
        pltpu.sync_copy(rows.at[pl.ds(0, 16)],
                        acc.at[pl.ds(NS * RPS, 16)])

    plsc.subcore_barrier()

    for r in range(2):
        wreg = s * 2 + r
        pltpu.sync_copy(counts_hbm.at[wreg], cntbuf)
        cnt = cntbuf[pl.ds(0, 16)][0]
        nch = lax.shift_right_logical(cnt + (CH - 1), 7)
        pltpu.sync_copy(srcc_hbm.at[wreg], src_v)
        pltpu.sync_copy(wc_hbm.at[wreg], wbuf)

        def _fetch(k, b, buf):
            pltpu.make_async_copy(dstc_hbm.at[wreg, pl.ds(k * CH, CH)],
                                  dbuf.at[b], msem).start()
            pltpu.make_async_copy(h1x_hbm.at[src_v.at[pl.ds(k * CH, CH)]],
                                  buf, gsem).start()

        def _wait(b, buf):
            pltpu.make_async_copy(dstc_hbm.at[wreg, pl.ds(0, CH)],
                                  dbuf.at[b], msem).wait()
            pltpu.make_async_copy(h1x_hbm.at[src_v.at[pl.ds(0, CH)]],
                                  buf, gsem).wait()

        def _scale_scatter(k, b, buf):
            @pl.loop(0, CH, step=16)
            def _scale(g):
                wgrp = wbuf[pl.ds(k * CH + g, 16)]
                for j in range(16):
                    wj = wgrp[j]
                    for t in range(8):
                        buf[g + j, pl.ds(t * 16, 16)] = (
                            buf[g + j, pl.ds(t * 16, 16)] * wj)

            pltpu.sync_copy(buf, acc.at[dbuf.at[b]], add=True)

        def _pair(i, carry):
            k = i * 2
            _fetch(k, 0, rows)

            @pl.when(k + 1 < nch)
            def _():
                _fetch(k + 1, 1, rows2)

            _wait(0, rows)
            _scale_scatter(k, 0, rows)

            @pl.when(k + 1 < nch)
            def _():
                _wait(1, rows2)
                _scale_scatter(k + 1, 1, rows2)

            return carry

        lax.fori_loop(0, lax.shift_right_logical(nch + 1, 1), _pair,
                      jnp.int32(0))

    plsc.subcore_barrier()
    off = 0
    for blk in WB_BLOCKS:
        rr = row0 + off
        pltpu.sync_copy(acc.at[pl.ds(rr, blk)], outx_hbm.at[pl.ds(rr, blk)])
        off += blk

    @pl.when(s == NS - 1)
    def _wtail():
        pltpu.sync_copy(acc.at[pl.ds(NS * RPS, 16)],
                        outx_hbm.at[pl.ds(NS * RPS, 16)])


def _sc2_body(h1a_hbm, h1b_hbm, srcc_hbm, dstc_hbm, wc_hbm, counts_hbm,
              z2a_hbm, z2b_hbm,
              src_v, wbuf, dbuf, cntbuf, rows, rows2, acc, gsem, msem):
    c = lax.axis_index("c")
    s = lax.axis_index("s")

    @pl.when(c == 0)
    def _():
        _sc2_half(h1a_hbm, z2a_hbm, srcc_hbm, dstc_hbm, wc_hbm, counts_hbm,
                  src_v, wbuf, dbuf, cntbuf, rows, rows2, acc, gsem, msem, s)

    @pl.when(c == 1)
    def _():
        _sc2_half(h1b_hbm, z2b_hbm, srcc_hbm, dstc_hbm, wc_hbm, counts_hbm,
                  src_v, wbuf, dbuf, cntbuf, rows, rows2, acc, gsem, msem, s)


def _sc2(h1a, h1b, srcc, dstc, wc, counts):
    kern = pl.kernel(
        _sc2_body,
        out_type=[
            jax.ShapeDtypeStruct((N, 128), jnp.float32),
            jax.ShapeDtypeStruct((N, 128), jnp.float32),
        ],
        mesh=_mesh,
        scratch_types=[
            pltpu.VMEM((REG,), jnp.int32),
            pltpu.VMEM((REG,), jnp.float32),
            pltpu.VMEM((2, CH), jnp.int32),
            pltpu.VMEM((16,), jnp.int32),
            pltpu.VMEM((CH, 128), jnp.float32),
            pltpu.VMEM((CH, 128), jnp.float32),
            pltpu.VMEM_SHARED((N, 128), jnp.float32),
            pltpu.SemaphoreType.DMA,
            pltpu.SemaphoreType.DMA,
        ],
        compiler_params=_sc_params,
    )
    return kern(h1a, h1b, srcc, dstc, wc, counts)


def _g_body(feats_ref, w1b_ref, out_ref):
    out_ref[...] = jax.lax.dot_general(
        feats_ref[...], w1b_ref[...], (((1,), (0,)), ((), ())),
        preferred_element_type=jnp.float32,
        precision=jax.lax.Precision.HIGHEST)


def _tc_g(feats, w1b):
    bn = 1000
    return pl.pallas_call(
        _g_body,
        grid=(N // bn,),
        in_specs=[
            pl.BlockSpec((bn, 512), lambda i: (i, 0)),
            pl.BlockSpec((512, 256), lambda i: (0, 0)),
        ],
        out_specs=pl.BlockSpec((bn, 256), lambda i: (i, 0)),
        out_shape=jax.ShapeDtypeStruct((N, 256), jnp.float32),
    )(feats, w1b)


def _b_body(agg_ref, colors_ref, g_ref, w0_ref, b0_ref, w1a_ref,
            h1a_ref, h1b_ref):
    agg = agg_ref[0, :, 0:3] + agg_ref[1, :, 0:3]
    h0 = jax.lax.dot_general(
        colors_ref[...], w0_ref[...], (((1,), (0,)), ((), ())),
        preferred_element_type=jnp.float32,
        precision=jax.lax.Precision.HIGHEST)
    r = jnp.maximum(agg + h0 + b0_ref[...], 0.0)
    h1 = jax.lax.dot_general(
        r, w1a_ref[...], (((1,), (0,)), ((), ())),
        preferred_element_type=jnp.float32,
        precision=jax.lax.Precision.HIGHEST) + g_ref[...]
    h1a_ref[...] = h1[:, :128]
    h1b_ref[...] = h1[:, 128:]


def _tc_b(agg0, colors, g, w0, b0_row, w1a):
    bn = 1000
    agg0_r = agg0.reshape(NC, N4 // 4, 4)
    return pl.pallas_call(
        _b_body,
        grid=(N // bn,),
        in_specs=[
            pl.BlockSpec((NC, bn, 4), lambda i: (0, i, 0)),
            pl.BlockSpec((bn, 3), lambda i: (i, 0)),
            pl.BlockSpec((bn, 256), lambda i: (i, 0)),
            pl.BlockSpec((3, 3), lambda i: (0, 0)),
            pl.BlockSpec((1, 3), lambda i: (0, 0)),
            pl.BlockSpec((3, 256), lambda i: (0, 0)),
        ],
        out_specs=[
            pl.BlockSpec((bn, 128), lambda i: (i, 0)),
            pl.BlockSpec((bn, 128), lambda i: (i, 0)),
        ],
        out_shape=[
            jax.ShapeDtypeStruct((N, 128), jnp.float32),
            jax.ShapeDtypeStruct((N, 128), jnp.float32),
        ],
    )(agg0_r, colors, g, w0, b0_row, w1a)


def _c_body(z2a_ref, z2b_ref, h1a_ref, h1b_ref, aspp_ref, b1_ref, mu_ref,
            out_ref):
    agg = jnp.concatenate([z2a_ref[...], z2b_ref[...]], axis=1)
    h1 = jnp.concatenate([h1a_ref[...], h1b_ref[...]], axis=1)
    z2 = 0.5 * (agg + h1 + b1_ref[...] + aspp_ref[...])
    mu = mu_ref[...]
    dot = jax.lax.dot_general(
        z2, mu, (((1,), (1,)), ((), ())),
        preferred_element_type=jnp.float32,
        precision=jax.lax.Precision.HIGHEST)
    z2sq = jnp.sum(z2 * z2, axis=1, keepdims=True)
    musq = jnp.sum(mu * mu, axis=1)[None, :]
    d2 = jnp.maximum(z2sq + musq - 2.0 * dot, 0.0)
    f2 = 1.0 / (1.0 + d2)
    fmax = jnp.max(f2, axis=1, keepdims=True)
    ef = jnp.exp(f2 - fmax)
    out_ref[...] = ef / jnp.sum(ef, axis=1, keepdims=True)


def _tc_c(z2a, z2b, h1a, h1b, aspp, b1_row, mu):
    bn = 1000
    return pl.pallas_call(
        _c_body,
        grid=(N // bn,),
        in_specs=[
            pl.BlockSpec((bn, 128), lambda i: (i, 0)),
            pl.BlockSpec((bn, 128), lambda i: (i, 0)),
            pl.BlockSpec((bn, 128), lambda i: (i, 0)),
            pl.BlockSpec((bn, 128), lambda i: (i, 0)),
            pl.BlockSpec((bn, 256), lambda i: (i, 0)),
            pl.BlockSpec((1, 256), lambda i: (0, 0)),
            pl.BlockSpec((K, 256), lambda i: (0, 0)),
        ],
        out_specs=pl.BlockSpec((bn, K), lambda i: (i, 0)),
        out_shape=jax.ShapeDtypeStruct((N, K), jnp.float32),
    )(z2a, z2b, h1a, h1b, aspp, b1_row, mu)


def kernel(nodes_color, probas, feats_pooled, pooled_aspp_feats, edges_nn,
           W0, b0, W1, b1, mu):
    src = edges_nn[:, 0]
    dst = edges_nn[:, 1]
    colors_flat = nodes_color.T.reshape(-1)
    w0_pad = jnp.pad(W0.reshape(-1), (0, 7))

    srcc, dstc, wc, counts, agg0 = _sc1(colors_flat, probas, src, dst, w0_pad)
    g = _tc_g(feats_pooled, W1[3:])
    h1a, h1b = _tc_b(agg0, nodes_color, g, W0, b0.reshape(1, 3), W1[:3])

    z2a, z2b = _sc2(h1a, h1b, srcc, dstc, wc, counts)

    return _tc_c(z2a, z2b, h1a, h1b, pooled_aspp_feats,
                 b1.reshape(1, 256), mu)
